# Initial kernel scaffold; baseline (speedup 1.0000x reference)
import jax
import jax.numpy as jnp
from jax.experimental import pallas as pl


def kernel(feat_map0, feat_map1, feat_map2, feat_map3, feat_map4, W_dod, b_dod, W_hsi0, b_hsi0, g_hsi0, be_hsi0, W_hsi1, b_hsi1, g_hsi1, be_hsi1):
    raise NotImplementedError("write your pallas kernel here")


# speedup vs baseline: 1.8925x; 1.8925x over previous
import jax
import jax.numpy as jnp
import numpy as np
from jax.experimental import pallas as pl
from jax.experimental.pallas import tpu as pltpu

C = 256
K = 100
HWS = [(128, 128), (64, 64), (32, 32), (16, 16), (8, 8)]
SIZES = [h * w for (h, w) in HWS]
OFFS = [0]
for _s in SIZES:
    OFFS.append(OFFS[-1] + _s)
N = OFFS[-1]
ROWS = 176
NPAD = ROWS * 128


def _pos_table_np():
    half = C // 2
    dim_t = np.arange(half, dtype=np.float64)
    temp = 10000.0 ** (2.0 * np.floor(dim_t / 2.0) / half)
    even = (np.arange(half) % 2 == 0)
    parts = []
    for (H, W) in HWS:
        y = (np.arange(H, dtype=np.float64) + 0.5) / H * (2.0 * np.pi)
        x = (np.arange(W, dtype=np.float64) + 0.5) / W * (2.0 * np.pi)
        py = y[:, None] / temp
        px = x[:, None] / temp
        py = np.where(even[None, :], np.sin(py), np.cos(py))
        px = np.where(even[None, :], np.sin(px), np.cos(px))
        t = np.concatenate(
            [np.broadcast_to(py[:, None, :], (H, W, half)),
             np.broadcast_to(px[None, :, :], (H, W, half))], axis=-1)
        parts.append(t.reshape(H * W, C))
    return np.concatenate(parts, axis=0).astype(np.float32)


def _box_table_np():
    parts = []
    for (H, W) in HWS:
        ys = (np.arange(H, dtype=np.float64) + 0.5) / H
        xs = (np.arange(W, dtype=np.float64) + 0.5) / W
        cy, cx = np.meshgrid(ys, xs, indexing="ij")
        w = np.full((H, W), 1.0 / W)
        h = np.full((H, W), 1.0 / H)
        parts.append(np.stack(
            [cx.ravel(), cy.ravel(), w.ravel(), h.ravel()], axis=-1))
    return np.concatenate(parts, axis=0).astype(np.float32).reshape(N, 1, 4)


_POS_TAB = _pos_table_np()
_BOX_TAB = _box_table_np()


def _pass_a_body(w_ref, b_ref, pos_ref, x_ref, prob_ref, paat_ref):
    w = w_ref[...]
    x = x_ref[0]
    y = jnp.dot(w, x, preferred_element_type=jnp.float32) + b_ref[0]
    prob_ref[0] = jax.nn.sigmoid(y)
    paat_ref[0, :, 0, :] = jnp.transpose(x, (1, 0)) + pos_ref[...]


def _pass_a_call(fm, w_row, b_vec, pos_tab, off):
    B, _, HW = fm.shape
    T = min(HW, 2048)
    grid = (B, HW // T)
    probs, paat = pl.pallas_call(
        _pass_a_body,
        grid=grid,
        in_specs=[
            pl.BlockSpec((1, C), lambda b, j: (0, 0)),
            pl.BlockSpec(memory_space=pltpu.SMEM),
            pl.BlockSpec((T, C), lambda b, j, off=off, T=T: (off // T + j, 0)),
            pl.BlockSpec((1, C, T), lambda b, j: (b, 0, j)),
        ],
        out_specs=[
            pl.BlockSpec((1, 1, T), lambda b, j: (b, 0, j)),
            pl.BlockSpec((1, T, 1, C), lambda b, j: (b, j, 0, 0)),
        ],
        out_shape=[
            jax.ShapeDtypeStruct((B, 1, HW), jnp.float32),
            jax.ShapeDtypeStruct((B, HW, 1, C), jnp.float32),
        ],
    )(w_row, b_vec, pos_tab, fm)
    return probs.reshape(B, HW), paat


def _select_body(p_ref, sel_ref, vals_ref):
    vals_ref[...] = p_ref[0]
    row_id = jax.lax.broadcasted_iota(jnp.int32, (ROWS, 128), 0)
    col_id = jax.lax.broadcasted_iota(jnp.int32, (ROWS, 128), 1)
    ids = row_id * 128 + col_id
    lane = jax.lax.broadcasted_iota(jnp.int32, (1, 128), 1)

    def body(k, acc):
        v = vals_ref[...]
        m = jnp.min(v)
        idx = jnp.min(jnp.where(v == m, ids, jnp.int32(2**30)))
        acc = jnp.where(lane == k, idx, acc)
        vals_ref[...] = jnp.where(ids == idx, jnp.float32(2.0), v)
        return acc

    acc0 = jnp.zeros((1, 128), jnp.int32)
    sel_ref[0] = jax.lax.fori_loop(0, K, body, acc0)


def _select_call(probs_pad):
    B = probs_pad.shape[0]
    return pl.pallas_call(
        _select_body,
        grid=(B,),
        in_specs=[pl.BlockSpec((1, ROWS, 128), lambda b: (b, 0, 0))],
        out_specs=pl.BlockSpec((1, 1, 128), lambda b: (b, 0, 0)),
        out_shape=jax.ShapeDtypeStruct((B, 1, 128), jnp.int32),
        scratch_shapes=[pltpu.VMEM((ROWS, 128), jnp.float32)],
    )(probs_pad).reshape(B, 128)


def _pass_c_body(sel_ref, p0, p1, p2, p3, p4, box_tab,
                 w0_ref, b0_ref, g0_ref, be0_ref,
                 w1_ref, b1_ref, g1_ref, be1_ref,
                 hid_ref, box_ref,
                 xs, bs, sem_f, sem_b):
    paats = (p0, p1, p2, p3, p4)
    nsel = 2 * K

    def start(j, _):
        b = j // K
        k = j - b * K
        gid = sel_ref[b, k]
        pltpu.make_async_copy(box_tab.at[gid], bs.at[j], sem_b).start()
        for lvl in range(5):
            in_lvl = jnp.logical_and(gid >= OFFS[lvl], gid < OFFS[lvl + 1])

            @pl.when(in_lvl)
            def _():
                p = gid - OFFS[lvl]
                pltpu.make_async_copy(
                    paats[lvl].at[b, p], xs.at[j], sem_f).start()
        return 0

    jax.lax.fori_loop(0, nsel, start, 0)

    def drain(j, _):
        pltpu.make_async_copy(box_tab.at[0], bs.at[j], sem_b).wait()
        pltpu.make_async_copy(p0.at[0, 0], xs.at[j], sem_f).wait()
        return 0

    jax.lax.fori_loop(0, nsel, drain, 0)

    h = xs[:, 0, :]
    for (w_ref, b_ref, g_ref, be_ref) in ((w0_ref, b0_ref, g0_ref, be0_ref),
                                          (w1_ref, b1_ref, g1_ref, be1_ref)):
        mu = jnp.mean(h, axis=-1, keepdims=True)
        var = jnp.mean((h - mu) ** 2, axis=-1, keepdims=True)
        xn = (h - mu) / jnp.sqrt(var + 1e-5) * g_ref[...] + be_ref[...]
        xr = jnp.maximum(xn, 0.0)
        h = h + jnp.dot(xr, w_ref[...],
                        preferred_element_type=jnp.float32) + b_ref[...]
    hid_ref[...] = h
    box_ref[...] = bs[:, 0, :]


def _pass_c_call(sel, paats, w0, b0, g0, be0, w1, b1, g1, be1, box_tab):
    nsel = 2 * K
    any_spec = pl.BlockSpec(memory_space=pl.ANY)
    vspec = pl.BlockSpec(memory_space=pltpu.VMEM)
    return pl.pallas_call(
        _pass_c_body,
        in_specs=[pl.BlockSpec(memory_space=pltpu.SMEM)]
                 + [any_spec] * 6
                 + [vspec] * 8,
        out_specs=[vspec, vspec],
        out_shape=[jax.ShapeDtypeStruct((nsel, C), jnp.float32),
                   jax.ShapeDtypeStruct((nsel, 4), jnp.float32)],
        scratch_shapes=[
            pltpu.VMEM((nsel, 1, C), jnp.float32),
            pltpu.VMEM((nsel, 1, 4), jnp.float32),
            pltpu.SemaphoreType.DMA,
            pltpu.SemaphoreType.DMA,
        ],
    )(sel, *paats, box_tab, w0, b0, g0, be0, w1, b1, g1, be1)


def kernel(feat_map0, feat_map1, feat_map2, feat_map3, feat_map4,
           W_dod, b_dod, W_hsi0, b_hsi0, g_hsi0, be_hsi0,
           W_hsi1, b_hsi1, g_hsi1, be_hsi1):
    B = feat_map0.shape[0]
    fms = [fm.reshape(B, C, -1) for fm in
           (feat_map0, feat_map1, feat_map2, feat_map3, feat_map4)]
    w_row = W_dod.reshape(1, C)
    b_vec = b_dod.reshape(1)
    pos_tab = jnp.asarray(_POS_TAB)
    box_tab = jnp.asarray(_BOX_TAB)

    parts = []
    paats = []
    for lvl, fm in enumerate(fms):
        probs_l, paat_l = _pass_a_call(fm, w_row, b_vec, pos_tab, OFFS[lvl])
        parts.append(probs_l)
        paats.append(paat_l)
    obj_probs = jnp.concatenate(parts, axis=1)

    probs_pad = jnp.concatenate(
        [obj_probs, jnp.full((B, NPAD - N), 2.0, jnp.float32)],
        axis=1).reshape(B, ROWS, 128)
    sel = _select_call(probs_pad)

    hid, box = _pass_c_call(
        sel, paats,
        W_hsi0, b_hsi0.reshape(1, C), g_hsi0.reshape(1, C),
        be_hsi0.reshape(1, C),
        W_hsi1, b_hsi1.reshape(1, C), g_hsi1.reshape(1, C),
        be_hsi1.reshape(1, C),
        box_tab)
    return (hid, box, obj_probs)
